# Initial kernel scaffold; baseline (speedup 1.0000x reference)
import jax
import jax.numpy as jnp
from jax.experimental import pallas as pl


def kernel(pred_bbox, cls_logits):
    raise NotImplementedError("write your pallas kernel here")


# speedup vs baseline: 156.6207x; 156.6207x over previous
import jax
import jax.numpy as jnp
from jax.experimental import pallas as pl
from jax.experimental.pallas import tpu as pltpu

_N_CLASS = 21
_NMS_THRESH = 0.3
_SCORE_THRESH = 0.05
_PRE_NMS = 2000
_N_PAD = 2048
_B = 128
_NB = _N_PAD // _B
_CW = 512
_C = _N_CLASS - 1


def _iou_gt(y1a, x1a, y2a, x2a, aa, y1b, x1b, y2b, x2b, ab):
    iy1 = jnp.maximum(y1a[:, :, None], y1b[:, None, :])
    ix1 = jnp.maximum(x1a[:, :, None], x1b[:, None, :])
    iy2 = jnp.minimum(y2a[:, :, None], y2b[:, None, :])
    ix2 = jnp.minimum(x2a[:, :, None], x2b[:, None, :])
    inter = jnp.maximum(iy2 - iy1, 0.0) * jnp.maximum(ix2 - ix1, 0.0)
    union = aa[:, :, None] + ab[:, None, :] - inter
    return (inter / jnp.maximum(union, 1e-10)) > _NMS_THRESH


def _nms_body(y1_ref, x1_ref, y2_ref, x2_ref, vals_ref,
              oy1_ref, ox1_ref, oy2_ref, ox2_ref, os_ref,
              active_ref, supp_ref):
    y1 = y1_ref[:, :]
    x1 = x1_ref[:, :]
    y2 = y2_ref[:, :]
    x2 = x2_ref[:, :]
    vals = vals_ref[:, :]
    area = jnp.maximum(y2 - y1, 0.0) * jnp.maximum(x2 - x1, 0.0)

    active_ref[:, :] = jnp.where(vals > _SCORE_THRESH, 1.0, 0.0)

    lane = jax.lax.broadcasted_iota(jnp.int32, (1, _B), 1)

    for b in range(_NB):
        s = slice(b * _B, (b + 1) * _B)
        yb1, xb1, yb2, xb2, ab = y1[:, s], x1[:, s], y2[:, s], x2[:, s], area[:, s]

        supp_ref[:, :, :] = jnp.where(
            _iou_gt(yb1, xb1, yb2, xb2, ab, yb1, xb1, yb2, xb2, ab), 1.0, 0.0)

        act0 = active_ref[:, s]

        def inner(i, act):
            row = supp_ref[:, pl.ds(i, 1), :].reshape(_C, _B)
            keep_i = jnp.max(jnp.where(lane == i, act, 0.0), axis=1,
                             keepdims=True)
            later = jnp.where(lane > i, 1.0, 0.0)
            return act * (1.0 - keep_i * row * later)

        kb = jax.lax.fori_loop(0, _B, inner, act0)

        oy1_ref[:, s] = yb1 * kb
        ox1_ref[:, s] = xb1 * kb
        oy2_ref[:, s] = yb2 * kb
        ox2_ref[:, s] = xb2 * kb
        os_ref[:, s] = vals[:, s] * kb

        if b < _NB - 1:
            for c0 in range(0, _N_PAD, _CW):
                cs = slice(c0, c0 + _CW)
                sc = jnp.where(
                    _iou_gt(yb1, xb1, yb2, xb2, ab,
                            y1[:, cs], x1[:, cs], y2[:, cs], x2[:, cs],
                            area[:, cs]), 1.0, 0.0)
                supp = jnp.max(sc * kb[:, :, None], axis=1)
                active_ref[:, cs] = active_ref[:, cs] * (1.0 - supp)


def kernel(pred_bbox, cls_logits):
    prob = jax.nn.softmax(cls_logits, axis=1)
    boxes_all = pred_bbox.reshape(-1, _N_CLASS, 4)

    p = prob[:, 1:].T
    masked = jnp.where(p > _SCORE_THRESH, p, -1.0)
    vals, idx = jax.lax.top_k(masked, _PRE_NMS)

    boxes_cls = jnp.moveaxis(boxes_all, 1, 0)[1:]
    b = jnp.take_along_axis(boxes_cls, idx[:, :, None], axis=1)

    pad = _N_PAD - _PRE_NMS
    vals_p = jnp.pad(vals, ((0, 0), (0, pad)), constant_values=-1.0)
    b_p = jnp.pad(b, ((0, 0), (0, pad), (0, 0)))

    y1 = b_p[:, :, 0]
    x1 = b_p[:, :, 1]
    y2 = b_p[:, :, 2]
    x2 = b_p[:, :, 3]

    shp = jax.ShapeDtypeStruct((_C, _N_PAD), jnp.float32)
    oy1, ox1, oy2, ox2, osc = pl.pallas_call(
        _nms_body,
        out_shape=(shp, shp, shp, shp, shp),
        scratch_shapes=[pltpu.VMEM((_C, _N_PAD), jnp.float32),
                        pltpu.VMEM((_C, _B, _B), jnp.float32)],
    )(y1, x1, y2, x2, vals_p)

    out = jnp.stack([oy1, ox1, oy2, ox2, osc], axis=-1)
    return out[:, :_PRE_NMS, :]


# speedup vs baseline: 188.2598x; 1.2020x over previous
import jax
import jax.numpy as jnp
from jax.experimental import pallas as pl
from jax.experimental.pallas import tpu as pltpu

_N_CLASS = 21
_NMS_THRESH = 0.3
_SCORE_THRESH = 0.05
_PRE_NMS = 2000
_N_PAD = 2048
_B = 128
_NB = _N_PAD // _B
_CW = 512
_C = _N_CLASS - 1


def _iou_gt(y1a, x1a, y2a, x2a, aa, y1b, x1b, y2b, x2b, ab):
    iy1 = jnp.maximum(y1a[:, :, None], y1b[:, None, :])
    ix1 = jnp.maximum(x1a[:, :, None], x1b[:, None, :])
    iy2 = jnp.minimum(y2a[:, :, None], y2b[:, None, :])
    ix2 = jnp.minimum(x2a[:, :, None], x2b[:, None, :])
    inter = jnp.maximum(iy2 - iy1, 0.0) * jnp.maximum(ix2 - ix1, 0.0)
    union = aa[:, :, None] + ab[:, None, :] - inter
    return (inter / jnp.maximum(union, 1e-10)) > _NMS_THRESH


def _nms_body(y1_ref, x1_ref, y2_ref, x2_ref, vals_ref,
              oy1_ref, ox1_ref, oy2_ref, ox2_ref, os_ref,
              active_ref):
    y1 = y1_ref[:, :]
    x1 = x1_ref[:, :]
    y2 = y2_ref[:, :]
    x2 = x2_ref[:, :]
    vals = vals_ref[:, :]
    area = jnp.maximum(y2 - y1, 0.0) * jnp.maximum(x2 - x1, 0.0)

    active_ref[:, :] = jnp.where(vals > _SCORE_THRESH, 1.0, 0.0)

    ltri = jnp.where(
        jax.lax.broadcasted_iota(jnp.int32, (1, _B, _B), 1)
        < jax.lax.broadcasted_iota(jnp.int32, (1, _B, _B), 2), 1.0, 0.0)

    for b in range(_NB):
        s = slice(b * _B, (b + 1) * _B)
        yb1, xb1, yb2, xb2, ab = y1[:, s], x1[:, s], y2[:, s], x2[:, s], area[:, s]

        sl = jnp.where(
            _iou_gt(yb1, xb1, yb2, xb2, ab, yb1, xb1, yb2, xb2, ab), 1.0, 0.0)
        sl = sl * ltri

        act0 = active_ref[:, s]

        def cond(carry):
            _, changed = carry
            return changed

        def body(carry):
            k, _ = carry
            overlap = jnp.max(sl * k[:, :, None], axis=1)
            k_new = act0 * (1.0 - overlap)
            return k_new, jnp.any(k_new != k)

        kb, _ = jax.lax.while_loop(cond, body, (act0, True))

        oy1_ref[:, s] = yb1 * kb
        ox1_ref[:, s] = xb1 * kb
        oy2_ref[:, s] = yb2 * kb
        ox2_ref[:, s] = xb2 * kb
        os_ref[:, s] = vals[:, s] * kb

        if b < _NB - 1:
            for c0 in range(0, _N_PAD, _CW):
                cs = slice(c0, c0 + _CW)
                sc = jnp.where(
                    _iou_gt(yb1, xb1, yb2, xb2, ab,
                            y1[:, cs], x1[:, cs], y2[:, cs], x2[:, cs],
                            area[:, cs]), 1.0, 0.0)
                supp = jnp.max(sc * kb[:, :, None], axis=1)
                active_ref[:, cs] = active_ref[:, cs] * (1.0 - supp)


def kernel(pred_bbox, cls_logits):
    prob = jax.nn.softmax(cls_logits, axis=1)
    boxes_all = pred_bbox.reshape(-1, _N_CLASS, 4)

    p = prob[:, 1:].T
    masked = jnp.where(p > _SCORE_THRESH, p, -1.0)
    vals, idx = jax.lax.top_k(masked, _PRE_NMS)

    boxes_cls = jnp.moveaxis(boxes_all, 1, 0)[1:]
    b = jnp.take_along_axis(boxes_cls, idx[:, :, None], axis=1)

    pad = _N_PAD - _PRE_NMS
    vals_p = jnp.pad(vals, ((0, 0), (0, pad)), constant_values=-1.0)
    b_p = jnp.pad(b, ((0, 0), (0, pad), (0, 0)))

    y1 = b_p[:, :, 0]
    x1 = b_p[:, :, 1]
    y2 = b_p[:, :, 2]
    x2 = b_p[:, :, 3]

    shp = jax.ShapeDtypeStruct((_C, _N_PAD), jnp.float32)
    oy1, ox1, oy2, ox2, osc = pl.pallas_call(
        _nms_body,
        out_shape=(shp, shp, shp, shp, shp),
        scratch_shapes=[pltpu.VMEM((_C, _N_PAD), jnp.float32)],
    )(y1, x1, y2, x2, vals_p)

    out = jnp.stack([oy1, ox1, oy2, ox2, osc], axis=-1)
    return out[:, :_PRE_NMS, :]


# speedup vs baseline: 209.5355x; 1.1130x over previous
import jax
import jax.numpy as jnp
from jax.experimental import pallas as pl
from jax.experimental.pallas import tpu as pltpu

_N_CLASS = 21
_NMS_THRESH = 0.3
_SCORE_THRESH = 0.05
_PRE_NMS = 2000
_N_PAD = 2048
_B = 128
_NB = _N_PAD // _B
_CW = 512
_C = _N_CLASS - 1


def _iou_gt(y1a, x1a, y2a, x2a, aa, y1b, x1b, y2b, x2b, ab):
    iy1 = jnp.maximum(y1a[:, :, None], y1b[:, None, :])
    ix1 = jnp.maximum(x1a[:, :, None], x1b[:, None, :])
    iy2 = jnp.minimum(y2a[:, :, None], y2b[:, None, :])
    ix2 = jnp.minimum(x2a[:, :, None], x2b[:, None, :])
    inter = jnp.maximum(iy2 - iy1, 0.0) * jnp.maximum(ix2 - ix1, 0.0)
    union = aa[:, :, None] + ab[:, None, :] - inter
    return (inter / jnp.maximum(union, 1e-10)) > _NMS_THRESH


def _nms_body(y1_ref, x1_ref, y2_ref, x2_ref, vals_ref,
              oy1_ref, ox1_ref, oy2_ref, ox2_ref, os_ref,
              active_ref):
    y1 = y1_ref[:, :]
    x1 = x1_ref[:, :]
    y2 = y2_ref[:, :]
    x2 = x2_ref[:, :]
    vals = vals_ref[:, :]
    area = jnp.maximum(y2 - y1, 0.0) * jnp.maximum(x2 - x1, 0.0)

    active_ref[:, :] = jnp.where(vals > _SCORE_THRESH, 1.0, 0.0)

    ltri = jnp.where(
        jax.lax.broadcasted_iota(jnp.int32, (1, _B, _B), 1)
        < jax.lax.broadcasted_iota(jnp.int32, (1, _B, _B), 2), 1.0, 0.0)

    for b in range(_NB):
        s = slice(b * _B, (b + 1) * _B)
        yb1, xb1, yb2, xb2, ab = y1[:, s], x1[:, s], y2[:, s], x2[:, s], area[:, s]

        sl = jnp.where(
            _iou_gt(yb1, xb1, yb2, xb2, ab, yb1, xb1, yb2, xb2, ab), 1.0, 0.0)
        sl = sl * ltri

        act0 = active_ref[:, s]

        def cond(carry):
            _, changed = carry
            return changed

        def body(carry):
            k, _ = carry
            overlap = jnp.max(sl * k[:, :, None], axis=1)
            k_new = act0 * (1.0 - overlap)
            return k_new, jnp.any(k_new != k)

        kb, _ = jax.lax.while_loop(cond, body, (act0, True))

        oy1_ref[:, s] = yb1 * kb
        ox1_ref[:, s] = xb1 * kb
        oy2_ref[:, s] = yb2 * kb
        ox2_ref[:, s] = xb2 * kb
        os_ref[:, s] = vals[:, s] * kb

        if b < _NB - 1:
            for c0 in range(((b + 1) * _B) // _CW * _CW, _N_PAD, _CW):
                cs = slice(c0, c0 + _CW)
                sc = jnp.where(
                    _iou_gt(yb1, xb1, yb2, xb2, ab,
                            y1[:, cs], x1[:, cs], y2[:, cs], x2[:, cs],
                            area[:, cs]), 1.0, 0.0)
                supp = jnp.max(sc * kb[:, :, None], axis=1)
                active_ref[:, cs] = active_ref[:, cs] * (1.0 - supp)


def kernel(pred_bbox, cls_logits):
    prob = jax.nn.softmax(cls_logits, axis=1)
    boxes_all = pred_bbox.reshape(-1, _N_CLASS, 4)

    p = prob[:, 1:].T
    masked = jnp.where(p > _SCORE_THRESH, p, -1.0)
    vals, idx = jax.lax.top_k(masked, _PRE_NMS)

    boxes_cls = jnp.moveaxis(boxes_all, 1, 0)[1:]
    b = jnp.take_along_axis(boxes_cls, idx[:, :, None], axis=1)

    pad = _N_PAD - _PRE_NMS
    vals_p = jnp.pad(vals, ((0, 0), (0, pad)), constant_values=-1.0)
    b_p = jnp.pad(b, ((0, 0), (0, pad), (0, 0)))

    y1 = b_p[:, :, 0]
    x1 = b_p[:, :, 1]
    y2 = b_p[:, :, 2]
    x2 = b_p[:, :, 3]

    shp = jax.ShapeDtypeStruct((_C, _N_PAD), jnp.float32)
    oy1, ox1, oy2, ox2, osc = pl.pallas_call(
        _nms_body,
        out_shape=(shp, shp, shp, shp, shp),
        scratch_shapes=[pltpu.VMEM((_C, _N_PAD), jnp.float32)],
    )(y1, x1, y2, x2, vals_p)

    out = jnp.stack([oy1, ox1, oy2, ox2, osc], axis=-1)
    return out[:, :_PRE_NMS, :]
